# Initial kernel scaffold; baseline (speedup 1.0000x reference)
import jax
import jax.numpy as jnp
from jax.experimental import pallas as pl


def kernel(x, edge_index, batch, gamma_feat, beta_feat, W_feat, b_feat, conv_gamma, conv_beta, conv_W, conv_b, fc_gamma, fc_beta, fc_W, fc_b, gamma_hidden, beta_hidden):
    raise NotImplementedError("write your pallas kernel here")


# speedup vs baseline: 9.7550x; 9.7550x over previous
import functools

import jax
import jax.numpy as jnp
from jax import lax
from jax.experimental import pallas as pl
from jax.experimental.pallas import tpu as pltpu
from jax.experimental.pallas import tpu_sc as plsc

N_NODES = 10000
N_EDGES = 320000
D = 128
NUM_GRAPHS = 128
NUM_CONV = 3
NUM_FC = 2
EPS = 1e-5

NC = 2
NS = 16
NW = NC * NS
EPT = N_EDGES // NW
CH = 80
NCHUNK = EPT // CH
NPAD = 10240
RPT = NPAD // NS
DEG_W = 128

_mesh = plsc.VectorSubcoreMesh(core_axis_name="c", subcore_axis_name="s")


@functools.partial(
    pl.kernel,
    mesh=_mesh,
    out_type=jax.ShapeDtypeStruct((NC, NPAD, DEG_W), jnp.float32),
    scratch_types=[
        pltpu.VMEM((CH,), jnp.int32),
        pltpu.VMEM((CH, DEG_W), jnp.float32),
        pltpu.VMEM_SHARED((NPAD, DEG_W), jnp.float32),
        pltpu.SemaphoreType.DMA,
    ],
)
def _deg_kernel(dst_hbm, ones_hbm, zeros_hbm, out_hbm, dstv, onesv, acc, sem):
    c = lax.axis_index("c")
    s = lax.axis_index("s")
    wid = c * NS + s

    row0 = s * RPT
    pltpu.sync_copy(ones_hbm, onesv)
    pltpu.sync_copy(zeros_hbm, acc.at[pl.ds(row0, RPT)])
    plsc.subcore_barrier()

    def body(i, carry):
        base = wid * EPT + i * CH
        pltpu.sync_copy(dst_hbm.at[pl.ds(base, CH)], dstv)
        pltpu.sync_copy(onesv, acc.at[dstv], add=True)
        return carry

    lax.fori_loop(0, NCHUNK, body, 0)
    plsc.subcore_barrier()

    for k in range(RPT // CH):
        pltpu.sync_copy(acc.at[pl.ds(row0 + k * CH, CH)], onesv)
        pltpu.sync_copy(onesv, out_hbm.at[c, pl.ds(row0 + k * CH, CH)])


@functools.partial(
    pl.kernel,
    mesh=_mesh,
    out_type=jax.ShapeDtypeStruct((NC, NPAD, D), jnp.float32),
    scratch_types=[
        pltpu.VMEM((CH,), jnp.int32),
        pltpu.VMEM((CH,), jnp.int32),
        pltpu.VMEM((CH, D), jnp.float32),
        pltpu.VMEM_SHARED((NPAD, D), jnp.float32),
        pltpu.SemaphoreType.DMA,
    ],
)
def _edge_kernel(src_hbm, dst_hbm, y_hbm, zeros_hbm, out_hbm, srcv, dstv, rows,
                 acc, sem):
    c = lax.axis_index("c")
    s = lax.axis_index("s")
    wid = c * NS + s

    row0 = s * RPT
    pltpu.sync_copy(zeros_hbm, acc.at[pl.ds(row0, RPT)])
    plsc.subcore_barrier()

    def body(i, carry):
        base = wid * EPT + i * CH
        pltpu.sync_copy(src_hbm.at[pl.ds(base, CH)], srcv)
        pltpu.sync_copy(dst_hbm.at[pl.ds(base, CH)], dstv)
        pltpu.async_copy(y_hbm.at[srcv], rows, sem).wait()
        pltpu.sync_copy(rows, acc.at[dstv], add=True)
        return carry

    lax.fori_loop(0, NCHUNK, body, 0)
    plsc.subcore_barrier()

    for k in range(RPT // CH):
        pltpu.sync_copy(acc.at[pl.ds(row0 + k * CH, CH)], rows)
        pltpu.sync_copy(rows, out_hbm.at[c, pl.ds(row0 + k * CH, CH)])


def _bn(x, g, b):
    mu = jnp.mean(x, axis=0, keepdims=True)
    xc = x - mu
    var = jnp.mean(xc * xc, axis=0, keepdims=True)
    return xc * lax.rsqrt(var + EPS) * g[None, :] + b[None, :]


def _pre_body(x_ref, g_ref, b_ref, W_ref, bb_ref, degw_ref, h_ref, dinv_ref):
    x = x_ref[...]
    xn = _bn(x, g_ref[...], b_ref[...])
    h = jnp.maximum(jnp.dot(xn, W_ref[...], preferred_element_type=jnp.float32)
                    + bb_ref[...][None, :], 0.0)
    h_ref[...] = h
    deg = degw_ref[0, :N_NODES, 0] + degw_ref[1, :N_NODES, 0] + 1.0
    dinv_ref[...] = lax.rsqrt(deg)[:, None]


def _pre_call(x, g, b, W, bb, degw):
    return pl.pallas_call(
        _pre_body,
        out_shape=(
            jax.ShapeDtypeStruct((N_NODES, D), jnp.float32),
            jax.ShapeDtypeStruct((N_NODES, 1), jnp.float32),
        ),
    )(x, g, b, W, bb, degw)


def _prep_body(h_ref, g_ref, b_ref, W_ref, dinv_ref, y_ref):
    xn = _bn(h_ref[...], g_ref[...], b_ref[...])
    y_ref[...] = dinv_ref[...] * jnp.dot(xn, W_ref[...],
                                         preferred_element_type=jnp.float32)


def _prep_call(h, g, b, W, dinv):
    return pl.pallas_call(
        _prep_body,
        out_shape=jax.ShapeDtypeStruct((N_NODES, D), jnp.float32),
    )(h, g, b, W, dinv)


def _combine_body(h_ref, y_ref, parts_ref, dinv_ref, b_ref, o_ref):
    srow = parts_ref[0, :N_NODES, :] + parts_ref[1, :N_NODES, :] + y_ref[...]
    o_ref[...] = h_ref[...] + jnp.maximum(
        dinv_ref[...] * srow + b_ref[...][None, :], 0.0)


def _combine_call(h, y, parts, dinv, b):
    return pl.pallas_call(
        _combine_body,
        out_shape=jax.ShapeDtypeStruct((N_NODES, D), jnp.float32),
    )(h, y, parts, dinv, b)


def _post_body(h_ref, batch_ref, fg_ref, fb_ref, fW_ref, fbb_ref,
               gh_ref, bh_ref, o_ref):
    batch = batch_ref[...]
    gid = lax.broadcasted_iota(jnp.int32, (N_NODES, NUM_GRAPHS), 1)
    onehot = (batch[:, None] == gid).astype(jnp.float32)
    p = lax.dot_general(onehot, h_ref[...], (((0,), (0,)), ((), ())),
                        preferred_element_type=jnp.float32)
    for i in range(NUM_FC):
        xn = _bn(p, fg_ref[i], fb_ref[i])
        p = jnp.maximum(jnp.dot(xn, fW_ref[i],
                                preferred_element_type=jnp.float32)
                        + fbb_ref[i][None, :], 0.0)
    o_ref[...] = _bn(p, gh_ref[...], bh_ref[...])


def _post_call(h, batch, fg, fb, fW, fbb, gh, bh):
    return pl.pallas_call(
        _post_body,
        out_shape=jax.ShapeDtypeStruct((NUM_GRAPHS, D), jnp.float32),
    )(h, batch, fg, fb, fW, fbb, gh, bh)


def kernel(x, edge_index, batch, gamma_feat, beta_feat, W_feat, b_feat,
           conv_gamma, conv_beta, conv_W, conv_b,
           fc_gamma, fc_beta, fc_W, fc_b, gamma_hidden, beta_hidden):
    src = edge_index[0].astype(jnp.int32)
    dst = edge_index[1].astype(jnp.int32)
    batch = batch.astype(jnp.int32)

    zrows = jnp.zeros((RPT, D), jnp.float32)
    degw = _deg_kernel(dst, jnp.ones((CH, DEG_W), jnp.float32), zrows)
    h, dinv = _pre_call(x, gamma_feat, beta_feat, W_feat, b_feat, degw)
    for i in range(NUM_CONV):
        y = _prep_call(h, conv_gamma[i], conv_beta[i], conv_W[i], dinv)
        parts = _edge_kernel(src, dst, y, zrows)
        h = _combine_call(h, y, parts, dinv, conv_b[i])
    return _post_call(h, batch, fc_gamma, fc_beta, fc_W, fc_b,
                      gamma_hidden, beta_hidden)


# speedup vs baseline: 12.9618x; 1.3287x over previous
import functools

import jax
import jax.numpy as jnp
from jax import lax
from jax.experimental import pallas as pl
from jax.experimental.pallas import tpu as pltpu
from jax.experimental.pallas import tpu_sc as plsc

N_NODES = 10000
N_EDGES = 320000
D = 128
NUM_GRAPHS = 128
NUM_CONV = 3
NUM_FC = 2
EPS = 1e-5

NC = 2
NS = 16
NW = NC * NS
EPT = N_EDGES // NW
CH = 80
NCHUNK = EPT // CH
NPAIR = NCHUNK // 2
NPAD = 10240
RPT = NPAD // NS

_mesh = plsc.VectorSubcoreMesh(core_axis_name="c", subcore_axis_name="s")


@functools.partial(
    pl.kernel,
    mesh=_mesh,
    out_type=jax.ShapeDtypeStruct((NC, NPAD, D), jnp.float32),
    scratch_types=[
        pltpu.VMEM((CH,), jnp.int32),
        pltpu.VMEM((CH,), jnp.int32),
        pltpu.VMEM((CH, D), jnp.float32),
        pltpu.VMEM_SHARED((NPAD, D), jnp.float32),
        pltpu.SemaphoreType.DMA,
        pltpu.SemaphoreType.DMA,
    ],
)
def _deg_kernel(dst_hbm, ones_hbm, zeros_hbm, out_hbm, dst0, dst1, onesv,
                acc, s0, s1):
    c = lax.axis_index("c")
    s = lax.axis_index("s")
    wid = c * NS + s

    row0 = s * RPT
    pltpu.sync_copy(ones_hbm, onesv)
    pltpu.sync_copy(zeros_hbm, acc.at[pl.ds(row0, RPT)])
    plsc.subcore_barrier()

    def body(k, carry):
        b0 = wid * EPT + (2 * k) * CH
        pltpu.sync_copy(dst_hbm.at[pl.ds(b0, CH)], dst0)
        cs0 = pltpu.async_copy(onesv, acc.at[dst0], s0, add=True)
        pltpu.sync_copy(dst_hbm.at[pl.ds(b0 + CH, CH)], dst1)
        cs1 = pltpu.async_copy(onesv, acc.at[dst1], s1, add=True)
        cs0.wait()
        cs1.wait()
        return carry

    lax.fori_loop(0, NPAIR, body, 0)
    base = wid * EPT + 2 * NPAIR * CH
    pltpu.sync_copy(dst_hbm.at[pl.ds(base, CH)], dst0)
    pltpu.sync_copy(onesv, acc.at[dst0], add=True)
    plsc.subcore_barrier()

    for k in range(RPT // CH):
        pltpu.sync_copy(acc.at[pl.ds(row0 + k * CH, CH)], onesv)
        pltpu.sync_copy(onesv, out_hbm.at[c, pl.ds(row0 + k * CH, CH)])


@functools.partial(
    pl.kernel,
    mesh=_mesh,
    out_type=jax.ShapeDtypeStruct((NC, NPAD, D), jnp.float32),
    scratch_types=[
        pltpu.VMEM((CH,), jnp.int32),
        pltpu.VMEM((CH,), jnp.int32),
        pltpu.VMEM((CH,), jnp.int32),
        pltpu.VMEM((CH,), jnp.int32),
        pltpu.VMEM((CH, D), jnp.float32),
        pltpu.VMEM((CH, D), jnp.float32),
        pltpu.VMEM_SHARED((NPAD, D), jnp.float32),
        pltpu.SemaphoreType.DMA,
        pltpu.SemaphoreType.DMA,
        pltpu.SemaphoreType.DMA,
        pltpu.SemaphoreType.DMA,
    ],
)
def _edge_kernel(src_hbm, dst_hbm, y_hbm, zeros_hbm, out_hbm,
                 src0, dst0, src1, dst1, rows0, rows1, acc, g0, g1, s0, s1):
    c = lax.axis_index("c")
    s = lax.axis_index("s")
    wid = c * NS + s

    row0 = s * RPT
    pltpu.sync_copy(zeros_hbm, acc.at[pl.ds(row0, RPT)])
    plsc.subcore_barrier()

    def body(k, carry):
        b0 = wid * EPT + (2 * k) * CH
        b1 = b0 + CH
        pltpu.sync_copy(src_hbm.at[pl.ds(b0, CH)], src0)
        pltpu.sync_copy(dst_hbm.at[pl.ds(b0, CH)], dst0)
        cg0 = pltpu.async_copy(y_hbm.at[src0], rows0, g0)
        pltpu.sync_copy(src_hbm.at[pl.ds(b1, CH)], src1)
        pltpu.sync_copy(dst_hbm.at[pl.ds(b1, CH)], dst1)
        cg1 = pltpu.async_copy(y_hbm.at[src1], rows1, g1)
        cg0.wait()
        cs0 = pltpu.async_copy(rows0, acc.at[dst0], s0, add=True)
        cg1.wait()
        cs1 = pltpu.async_copy(rows1, acc.at[dst1], s1, add=True)
        cs0.wait()
        cs1.wait()
        return carry

    lax.fori_loop(0, NPAIR, body, 0)
    base = wid * EPT + 2 * NPAIR * CH
    pltpu.sync_copy(src_hbm.at[pl.ds(base, CH)], src0)
    pltpu.sync_copy(dst_hbm.at[pl.ds(base, CH)], dst0)
    pltpu.async_copy(y_hbm.at[src0], rows0, g0).wait()
    pltpu.sync_copy(rows0, acc.at[dst0], add=True)
    plsc.subcore_barrier()

    for k in range(RPT // CH):
        pltpu.sync_copy(acc.at[pl.ds(row0 + k * CH, CH)], rows0)
        pltpu.sync_copy(rows0, out_hbm.at[c, pl.ds(row0 + k * CH, CH)])


def _bn(x, g, b):
    mu = jnp.mean(x, axis=0, keepdims=True)
    xc = x - mu
    var = jnp.mean(xc * xc, axis=0, keepdims=True)
    return xc * lax.rsqrt(var + EPS) * g[None, :] + b[None, :]


def _pre_body(x_ref, g_ref, b_ref, W_ref, bb_ref, degw_ref, h_ref, dinv_ref):
    x = x_ref[...]
    xn = _bn(x, g_ref[...], b_ref[...])
    h = jnp.maximum(jnp.dot(xn, W_ref[...], preferred_element_type=jnp.float32)
                    + bb_ref[...][None, :], 0.0)
    h_ref[...] = h
    deg = degw_ref[0, :N_NODES, 0] + degw_ref[1, :N_NODES, 0] + 1.0
    dinv_ref[...] = lax.rsqrt(deg)[:, None]


def _pre_call(x, g, b, W, bb, degw):
    return pl.pallas_call(
        _pre_body,
        out_shape=(
            jax.ShapeDtypeStruct((N_NODES, D), jnp.float32),
            jax.ShapeDtypeStruct((N_NODES, 1), jnp.float32),
        ),
    )(x, g, b, W, bb, degw)


def _prep_body(h_ref, g_ref, b_ref, W_ref, dinv_ref, y_ref):
    xn = _bn(h_ref[...], g_ref[...], b_ref[...])
    y_ref[...] = dinv_ref[...] * jnp.dot(xn, W_ref[...],
                                         preferred_element_type=jnp.float32)


def _prep_call(h, g, b, W, dinv):
    return pl.pallas_call(
        _prep_body,
        out_shape=jax.ShapeDtypeStruct((N_NODES, D), jnp.float32),
    )(h, g, b, W, dinv)


def _combine_body(h_ref, y_ref, parts_ref, dinv_ref, b_ref, o_ref):
    srow = parts_ref[0, :N_NODES, :] + parts_ref[1, :N_NODES, :] + y_ref[...]
    o_ref[...] = h_ref[...] + jnp.maximum(
        dinv_ref[...] * srow + b_ref[...][None, :], 0.0)


def _combine_call(h, y, parts, dinv, b):
    return pl.pallas_call(
        _combine_body,
        out_shape=jax.ShapeDtypeStruct((N_NODES, D), jnp.float32),
    )(h, y, parts, dinv, b)


def _post_body(h_ref, batch_ref, fg_ref, fb_ref, fW_ref, fbb_ref,
               gh_ref, bh_ref, o_ref):
    batch = batch_ref[...]
    gid = lax.broadcasted_iota(jnp.int32, (N_NODES, NUM_GRAPHS), 1)
    onehot = (batch[:, None] == gid).astype(jnp.float32)
    p = lax.dot_general(onehot, h_ref[...], (((0,), (0,)), ((), ())),
                        preferred_element_type=jnp.float32)
    for i in range(NUM_FC):
        xn = _bn(p, fg_ref[i], fb_ref[i])
        p = jnp.maximum(jnp.dot(xn, fW_ref[i],
                                preferred_element_type=jnp.float32)
                        + fbb_ref[i][None, :], 0.0)
    o_ref[...] = _bn(p, gh_ref[...], bh_ref[...])


def _post_call(h, batch, fg, fb, fW, fbb, gh, bh):
    return pl.pallas_call(
        _post_body,
        out_shape=jax.ShapeDtypeStruct((NUM_GRAPHS, D), jnp.float32),
    )(h, batch, fg, fb, fW, fbb, gh, bh)


def kernel(x, edge_index, batch, gamma_feat, beta_feat, W_feat, b_feat,
           conv_gamma, conv_beta, conv_W, conv_b,
           fc_gamma, fc_beta, fc_W, fc_b, gamma_hidden, beta_hidden):
    src = edge_index[0].astype(jnp.int32)
    dst = edge_index[1].astype(jnp.int32)
    batch = batch.astype(jnp.int32)

    zrows = jnp.zeros((RPT, D), jnp.float32)
    degw = _deg_kernel(dst, jnp.ones((CH, D), jnp.float32), zrows)
    h, dinv = _pre_call(x, gamma_feat, beta_feat, W_feat, b_feat, degw)
    for i in range(NUM_CONV):
        y = _prep_call(h, conv_gamma[i], conv_beta[i], conv_W[i], dinv)
        parts = _edge_kernel(src, dst, y, zrows)
        h = _combine_call(h, y, parts, dinv, conv_b[i])
    return _post_call(h, batch, fc_gamma, fc_beta, fc_W, fc_b,
                      gamma_hidden, beta_hidden)


# speedup vs baseline: 17.6519x; 1.3618x over previous
import functools

import jax
import jax.numpy as jnp
from jax import lax
from jax.experimental import pallas as pl
from jax.experimental.pallas import tpu as pltpu
from jax.experimental.pallas import tpu_sc as plsc

N_NODES = 10000
N_EDGES = 320000
D = 128
NUM_GRAPHS = 128
NUM_CONV = 3
NUM_FC = 2
EPS = 1e-5

NC = 2
NS = 16
NW = NC * NS
EPT = N_EDGES // NW
CH = 128
NPAIR = 39
CHT = EPT - 2 * NPAIR * CH
NPAD = 10240
RPT = NPAD // NS

_mesh = plsc.VectorSubcoreMesh(core_axis_name="c", subcore_axis_name="s")


@functools.partial(
    pl.kernel,
    mesh=_mesh,
    out_type=jax.ShapeDtypeStruct((NC, NPAD, D), jnp.float32),
    scratch_types=[
        pltpu.VMEM((CH,), jnp.int32),
        pltpu.VMEM((CH,), jnp.int32),
        pltpu.VMEM((CHT,), jnp.int32),
        pltpu.VMEM((CH, D), jnp.float32),
        pltpu.VMEM_SHARED((NPAD, D), jnp.float32),
        pltpu.SemaphoreType.DMA,
        pltpu.SemaphoreType.DMA,
    ],
)
def _deg_kernel(dst_hbm, ones_hbm, zeros_hbm, out_hbm, dst0, dst1, dstt, onesv,
                acc, s0, s1):
    c = lax.axis_index("c")
    s = lax.axis_index("s")
    wid = c * NS + s

    row0 = s * RPT
    pltpu.sync_copy(ones_hbm, onesv)
    pltpu.sync_copy(zeros_hbm, acc.at[pl.ds(row0, RPT)])
    plsc.subcore_barrier()

    def body(k, carry):
        b0 = wid * EPT + (2 * k) * CH
        pltpu.sync_copy(dst_hbm.at[pl.ds(b0, CH)], dst0)
        cs0 = pltpu.async_copy(onesv, acc.at[dst0], s0, add=True)
        pltpu.sync_copy(dst_hbm.at[pl.ds(b0 + CH, CH)], dst1)
        cs1 = pltpu.async_copy(onesv, acc.at[dst1], s1, add=True)
        cs0.wait()
        cs1.wait()
        return carry

    lax.fori_loop(0, NPAIR, body, 0)
    base = wid * EPT + 2 * NPAIR * CH
    pltpu.sync_copy(dst_hbm.at[pl.ds(base, CHT)], dstt)
    pltpu.sync_copy(onesv.at[pl.ds(0, CHT)], acc.at[dstt], add=True)
    plsc.subcore_barrier()

    for k in range(RPT // CH):
        pltpu.sync_copy(acc.at[pl.ds(row0 + k * CH, CH)], onesv)
        pltpu.sync_copy(onesv, out_hbm.at[c, pl.ds(row0 + k * CH, CH)])


@functools.partial(
    pl.kernel,
    mesh=_mesh,
    out_type=jax.ShapeDtypeStruct((NC, NPAD, D), jnp.float32),
    scratch_types=[
        pltpu.VMEM((EPT,), jnp.int32),
        pltpu.VMEM((CH,), jnp.int32),
        pltpu.VMEM((CH,), jnp.int32),
        pltpu.VMEM((CHT,), jnp.int32),
        pltpu.VMEM((CH, D), jnp.float32),
        pltpu.VMEM((CH, D), jnp.float32),
        pltpu.VMEM_SHARED((NPAD, D), jnp.float32),
        pltpu.SemaphoreType.DMA,
        pltpu.SemaphoreType.DMA,
        pltpu.SemaphoreType.DMA,
        pltpu.SemaphoreType.DMA,
    ],
)
def _edge_kernel(src_hbm, dst_hbm, y_hbm, zeros_hbm, out_hbm,
                 srcall, dst0, dst1, dstt, rows0, rows1, acc, g0, g1, s0, s1):
    c = lax.axis_index("c")
    s = lax.axis_index("s")
    wid = c * NS + s

    row0 = s * RPT
    pltpu.sync_copy(src_hbm.at[pl.ds(wid * EPT, EPT)], srcall)
    pltpu.sync_copy(zeros_hbm, acc.at[pl.ds(row0, RPT)])
    plsc.subcore_barrier()

    def body(k, carry):
        b0 = wid * EPT + (2 * k) * CH
        e0 = (2 * k) * CH
        cg0 = pltpu.async_copy(y_hbm.at[srcall.at[pl.ds(e0, CH)]], rows0, g0)
        cg1 = pltpu.async_copy(y_hbm.at[srcall.at[pl.ds(e0 + CH, CH)]], rows1, g1)
        pltpu.sync_copy(dst_hbm.at[pl.ds(b0, CH)], dst0)
        pltpu.sync_copy(dst_hbm.at[pl.ds(b0 + CH, CH)], dst1)
        cg0.wait()
        cs0 = pltpu.async_copy(rows0, acc.at[dst0], s0, add=True)
        cg1.wait()
        cs1 = pltpu.async_copy(rows1, acc.at[dst1], s1, add=True)
        cs0.wait()
        cs1.wait()
        return carry

    lax.fori_loop(0, NPAIR, body, 0)
    base = wid * EPT + 2 * NPAIR * CH
    pltpu.sync_copy(dst_hbm.at[pl.ds(base, CHT)], dstt)
    pltpu.async_copy(y_hbm.at[srcall.at[pl.ds(2 * NPAIR * CH, CHT)]],
                     rows0.at[pl.ds(0, CHT)], g0).wait()
    pltpu.sync_copy(rows0.at[pl.ds(0, CHT)], acc.at[dstt], add=True)
    plsc.subcore_barrier()

    for k in range(RPT // CH):
        pltpu.sync_copy(acc.at[pl.ds(row0 + k * CH, CH)], rows0)
        pltpu.sync_copy(rows0, out_hbm.at[c, pl.ds(row0 + k * CH, CH)])


def _bn(x, g, b):
    mu = jnp.mean(x, axis=0, keepdims=True)
    xc = x - mu
    var = jnp.mean(xc * xc, axis=0, keepdims=True)
    return xc * lax.rsqrt(var + EPS) * g[None, :] + b[None, :]


def _pre_body(x_ref, g_ref, b_ref, W_ref, bb_ref, degw_ref, h_ref, dinv_ref):
    x = x_ref[...]
    xn = _bn(x, g_ref[...], b_ref[...])
    h = jnp.maximum(jnp.dot(xn, W_ref[...], preferred_element_type=jnp.float32)
                    + bb_ref[...][None, :], 0.0)
    h_ref[...] = h
    deg = degw_ref[0, :N_NODES, 0] + degw_ref[1, :N_NODES, 0] + 1.0
    dinv_ref[...] = lax.rsqrt(deg)[:, None]


def _pre_call(x, g, b, W, bb, degw):
    return pl.pallas_call(
        _pre_body,
        out_shape=(
            jax.ShapeDtypeStruct((N_NODES, D), jnp.float32),
            jax.ShapeDtypeStruct((N_NODES, 1), jnp.float32),
        ),
    )(x, g, b, W, bb, degw)


def _prep_body(h_ref, g_ref, b_ref, W_ref, dinv_ref, y_ref):
    xn = _bn(h_ref[...], g_ref[...], b_ref[...])
    y_ref[...] = dinv_ref[...] * jnp.dot(xn, W_ref[...],
                                         preferred_element_type=jnp.float32)


def _prep_call(h, g, b, W, dinv):
    return pl.pallas_call(
        _prep_body,
        out_shape=jax.ShapeDtypeStruct((N_NODES, D), jnp.float32),
    )(h, g, b, W, dinv)


def _combine_body(h_ref, y_ref, parts_ref, dinv_ref, b_ref, o_ref):
    srow = parts_ref[0, :N_NODES, :] + parts_ref[1, :N_NODES, :] + y_ref[...]
    o_ref[...] = h_ref[...] + jnp.maximum(
        dinv_ref[...] * srow + b_ref[...][None, :], 0.0)


def _combine_call(h, y, parts, dinv, b):
    return pl.pallas_call(
        _combine_body,
        out_shape=jax.ShapeDtypeStruct((N_NODES, D), jnp.float32),
    )(h, y, parts, dinv, b)


def _post_body(h_ref, batch_ref, fg_ref, fb_ref, fW_ref, fbb_ref,
               gh_ref, bh_ref, o_ref):
    batch = batch_ref[...]
    gid = lax.broadcasted_iota(jnp.int32, (N_NODES, NUM_GRAPHS), 1)
    onehot = (batch[:, None] == gid).astype(jnp.float32)
    p = lax.dot_general(onehot, h_ref[...], (((0,), (0,)), ((), ())),
                        preferred_element_type=jnp.float32)
    for i in range(NUM_FC):
        xn = _bn(p, fg_ref[i], fb_ref[i])
        p = jnp.maximum(jnp.dot(xn, fW_ref[i],
                                preferred_element_type=jnp.float32)
                        + fbb_ref[i][None, :], 0.0)
    o_ref[...] = _bn(p, gh_ref[...], bh_ref[...])


def _post_call(h, batch, fg, fb, fW, fbb, gh, bh):
    return pl.pallas_call(
        _post_body,
        out_shape=jax.ShapeDtypeStruct((NUM_GRAPHS, D), jnp.float32),
    )(h, batch, fg, fb, fW, fbb, gh, bh)


def kernel(x, edge_index, batch, gamma_feat, beta_feat, W_feat, b_feat,
           conv_gamma, conv_beta, conv_W, conv_b,
           fc_gamma, fc_beta, fc_W, fc_b, gamma_hidden, beta_hidden):
    src = edge_index[0].astype(jnp.int32)
    dst = edge_index[1].astype(jnp.int32)
    batch = batch.astype(jnp.int32)

    zrows = jnp.zeros((RPT, D), jnp.float32)
    degw = _deg_kernel(dst, jnp.ones((CH, D), jnp.float32), zrows)
    h, dinv = _pre_call(x, gamma_feat, beta_feat, W_feat, b_feat, degw)
    for i in range(NUM_CONV):
        y = _prep_call(h, conv_gamma[i], conv_beta[i], conv_W[i], dinv)
        parts = _edge_kernel(src, dst, y, zrows)
        h = _combine_call(h, y, parts, dinv, conv_b[i])
    return _post_call(h, batch, fc_gamma, fc_beta, fc_W, fc_b,
                      gamma_hidden, beta_hidden)
